# baseline (speedup 1.0000x reference)
import jax
import jax.numpy as jnp
import numpy as np
from jax.experimental import pallas as pl
from jax.experimental.pallas import tpu as pltpu

_S = 56
_KS = 10.0
_BB = 256


def _sigmoid(z):
    return 1.0 / (1.0 + jnp.exp(-z))


def _body(x_ref, W1_ref, b1_ref, W2_ref, b2_ref, e1_ref, e2_ref, o_ref):
    h = jnp.dot(x_ref[...], W1_ref[...], preferred_element_type=jnp.float32)
    h = h + b1_ref[...]
    t = jnp.dot(h, W2_ref[...], preferred_element_type=jnp.float32)
    t = t + b2_ref[...]
    t0 = t[:, 0:1]
    t1 = t[:, 1:2]
    half = 0.5 * t[:, 2:3]
    idx = jax.lax.broadcasted_iota(jnp.int32, (_BB, _S), 1).astype(jnp.float32)
    vx = _sigmoid(_KS * (idx - t0 + half)) - _sigmoid(_KS * (idx - t0 - half))
    vy = _sigmoid(_KS * (idx - t1 + half)) - _sigmoid(_KS * (idx - t1 - half))
    ex = jnp.dot(vx.astype(jnp.bfloat16), e1_ref[...],
                 preferred_element_type=jnp.float32)
    ey = jnp.dot(vy.astype(jnp.bfloat16), e2_ref[...],
                 preferred_element_type=jnp.float32)
    o_ref[...] = ex * ey


def kernel(x, W1, b1, W2, b2):
    B, D = x.shape
    H = W1.shape[1]
    S2 = _S * _S

    W2p = jnp.zeros((H, 128), W2.dtype).at[:, :3].set(W2)
    b2p = jnp.zeros((1, 128), b2.dtype).at[:, :3].set(b2)
    b1r = b1.reshape(1, H)

    l = np.arange(S2)
    e1 = jnp.asarray((l[None, :] // _S) == np.arange(_S)[:, None],
                     dtype=jnp.bfloat16)
    e2 = jnp.asarray((l[None, :] % _S) == np.arange(_S)[:, None],
                     dtype=jnp.bfloat16)

    out2 = pl.pallas_call(
        _body,
        out_shape=jax.ShapeDtypeStruct((B, S2), jnp.float32),
        grid=(B // _BB,),
        in_specs=[
            pl.BlockSpec((_BB, D), lambda i: (i, 0)),
            pl.BlockSpec((D, H), lambda i: (0, 0)),
            pl.BlockSpec((1, H), lambda i: (0, 0)),
            pl.BlockSpec((H, 128), lambda i: (0, 0)),
            pl.BlockSpec((1, 128), lambda i: (0, 0)),
            pl.BlockSpec((_S, S2), lambda i: (0, 0)),
            pl.BlockSpec((_S, S2), lambda i: (0, 0)),
        ],
        out_specs=pl.BlockSpec((_BB, S2), lambda i: (i, 0)),
        compiler_params=pltpu.CompilerParams(
            dimension_semantics=("parallel",),
            vmem_limit_bytes=100 * 1024 * 1024,
        ),
        name="rcn_fused",
    )(x, W1, b1r, W2p, b2p, e1, e2)
    return out2.reshape(B, _S, _S)


# speedup vs baseline: 3.3662x; 3.3662x over previous
import jax
import jax.numpy as jnp
from jax.experimental import pallas as pl
from jax.experimental.pallas import tpu as pltpu

_S = 56
_KS = 10.0
_BB = 512


def _sigmoid(z):
    return 1.0 / (1.0 + jnp.exp(-z))


def _body(x_ref, W1_ref, b1_ref, W2_ref, b2_ref, o_ref):
    h = jnp.dot(x_ref[...], W1_ref[...], preferred_element_type=jnp.float32)
    h = h + b1_ref[...]
    t = jnp.dot(h, W2_ref[...], preferred_element_type=jnp.float32)
    t = t + b2_ref[...]
    tT = t.T
    t0 = tT[0:1, :]
    t1 = tT[1:2, :]
    half = 0.5 * tT[2:3, :]
    ii = jax.lax.broadcasted_iota(jnp.int32, (_S, _BB), 0).astype(jnp.float32)
    vx = _sigmoid(_KS * (ii - t0 + half)) - _sigmoid(_KS * (ii - t0 - half))
    vy = _sigmoid(_KS * (ii - t1 + half)) - _sigmoid(_KS * (ii - t1 - half))
    o_ref[...] = vx[:, None, :] * vy[None, :, :]


def kernel(x, W1, b1, W2, b2):
    B, D = x.shape
    H = W1.shape[1]

    W2p = jnp.zeros((H, 128), W2.dtype).at[:, :3].set(W2)
    b2p = jnp.zeros((1, 128), b2.dtype).at[:, :3].set(b2)
    b1r = b1.reshape(1, H)

    outT = pl.pallas_call(
        _body,
        out_shape=jax.ShapeDtypeStruct((_S, _S, B), jnp.float32),
        grid=(B // _BB,),
        in_specs=[
            pl.BlockSpec((_BB, D), lambda i: (i, 0)),
            pl.BlockSpec((D, H), lambda i: (0, 0)),
            pl.BlockSpec((1, H), lambda i: (0, 0)),
            pl.BlockSpec((H, 128), lambda i: (0, 0)),
            pl.BlockSpec((1, 128), lambda i: (0, 0)),
        ],
        out_specs=pl.BlockSpec((_S, _S, _BB), lambda i: (0, 0, i)),
        compiler_params=pltpu.CompilerParams(
            dimension_semantics=("arbitrary",),
            vmem_limit_bytes=100 * 1024 * 1024,
        ),
        name="rcn_fused",
    )(x, W1, b1r, W2p, b2p)
    return jnp.transpose(outT, (2, 0, 1))


# speedup vs baseline: 3.5890x; 1.0662x over previous
import jax
import jax.numpy as jnp
from jax.experimental import pallas as pl
from jax.experimental.pallas import tpu as pltpu

_S = 56
_KS = 10.0
_BB = 1024


def _sigmoid(z):
    return 1.0 / (1.0 + jnp.exp(-z))


def _body(x_ref, W1_ref, b1_ref, W2_ref, b2_ref, o_ref):
    h = jnp.dot(x_ref[...], W1_ref[...], preferred_element_type=jnp.float32)
    h = h + b1_ref[...]
    t = jnp.dot(h, W2_ref[...], preferred_element_type=jnp.float32)
    t = t + b2_ref[...]
    tT = t.T
    t0 = tT[0:1, :]
    t1 = tT[1:2, :]
    half = 0.5 * tT[2:3, :]
    ii = jax.lax.broadcasted_iota(jnp.int32, (_S, _BB), 0).astype(jnp.float32)
    vx = _sigmoid(_KS * (ii - t0 + half)) - _sigmoid(_KS * (ii - t0 - half))
    vy = _sigmoid(_KS * (ii - t1 + half)) - _sigmoid(_KS * (ii - t1 - half))
    o_ref[...] = vx[:, None, :] * vy[None, :, :]


def kernel(x, W1, b1, W2, b2):
    B, D = x.shape
    H = W1.shape[1]

    W2p = jnp.zeros((H, 128), W2.dtype).at[:, :3].set(W2)
    b2p = jnp.zeros((1, 128), b2.dtype).at[:, :3].set(b2)
    b1r = b1.reshape(1, H)

    outT = pl.pallas_call(
        _body,
        out_shape=jax.ShapeDtypeStruct((_S, _S, B), jnp.float32),
        grid=(B // _BB,),
        in_specs=[
            pl.BlockSpec((_BB, D), lambda i: (i, 0)),
            pl.BlockSpec((D, H), lambda i: (0, 0)),
            pl.BlockSpec((1, H), lambda i: (0, 0)),
            pl.BlockSpec((H, 128), lambda i: (0, 0)),
            pl.BlockSpec((1, 128), lambda i: (0, 0)),
        ],
        out_specs=pl.BlockSpec((_S, _S, _BB), lambda i: (0, 0, i)),
        compiler_params=pltpu.CompilerParams(
            dimension_semantics=("arbitrary",),
            vmem_limit_bytes=58 * 1024 * 1024,
        ),
        name="rcn_fused",
    )(x, W1, b1r, W2p, b2p)
    return jnp.transpose(outT, (2, 0, 1))


# speedup vs baseline: 3.7135x; 1.0347x over previous
import jax
import jax.numpy as jnp
from jax.experimental import pallas as pl
from jax.experimental.pallas import tpu as pltpu

_S = 56
_KS = 10.0
_BB = 1024


def _sigmoid(z):
    return 1.0 / (1.0 + jnp.exp(-z))


def _body(x_ref, W1_ref, b1_ref, W2_ref, b2_ref, o_ref):
    h = jnp.dot(x_ref[...], W1_ref[...], preferred_element_type=jnp.float32)
    h = h + b1_ref[...]
    t = jnp.dot(h, W2_ref[...], preferred_element_type=jnp.float32)
    t = t + b2_ref[...]
    tT = t.T
    t0 = tT[0:1, :]
    t1 = tT[1:2, :]
    half = 0.5 * tT[2:3, :]
    ii = jax.lax.broadcasted_iota(jnp.int32, (_S, _BB), 0).astype(jnp.float32)
    vx = _sigmoid(_KS * (ii - t0 + half)) - _sigmoid(_KS * (ii - t0 - half))
    vy = _sigmoid(_KS * (ii - t1 + half)) - _sigmoid(_KS * (ii - t1 - half))
    o_ref[...] = vx[:, None, :] * vy[None, :, :]


def kernel(x, W1, b1, W2, b2):
    B, D = x.shape
    H = W1.shape[1]

    b1r = b1.reshape(1, H)
    b2r = b2.reshape(1, 3)

    outT = pl.pallas_call(
        _body,
        out_shape=jax.ShapeDtypeStruct((_S, _S, B), jnp.float32),
        grid=(B // _BB,),
        in_specs=[
            pl.BlockSpec((_BB, D), lambda i: (i, 0)),
            pl.BlockSpec((D, H), lambda i: (0, 0)),
            pl.BlockSpec((1, H), lambda i: (0, 0)),
            pl.BlockSpec((H, 3), lambda i: (0, 0)),
            pl.BlockSpec((1, 3), lambda i: (0, 0)),
        ],
        out_specs=pl.BlockSpec((_S, _S, _BB), lambda i: (0, 0, i)),
        compiler_params=pltpu.CompilerParams(
            dimension_semantics=("arbitrary",),
            vmem_limit_bytes=58 * 1024 * 1024,
        ),
        name="rcn_fused",
    )(x, W1, b1r, W2, b2r)
    return jnp.transpose(outT, (2, 0, 1))
